# baseline (speedup 1.0000x reference)
import functools

import jax
import jax.numpy as jnp
from jax import lax
from jax.experimental import pallas as pl
from jax.experimental.pallas import tpu as pltpu
from jax.experimental.pallas import tpu_sc as plsc

N = 10000
E = 320000
D_IN = 128
D_HID = 16
D_OUT = 2

NC = 2
NS = 16
NW = NC * NS
NPAD = 10240
ROWS_PER_TILE = NPAD // NS
K = 128
TOTAL_CHUNKS = 2560
EPAD = TOTAL_CHUNKS * K
CH_A = 96
CH_B = 64
CH_MAX = 96
NBUF = 16
DG = 16

def _chunk_span(cid, sid):
    base = lax.select(cid == 0, sid * CH_A, 16 * CH_A + sid * CH_B)
    load_base = jnp.minimum(base, TOTAL_CHUNKS - CH_MAX)
    trips = lax.select(cid == 0, CH_A, CH_B)
    return load_base, base - load_base, trips


def _sc_degree_body(dst_hbm, out_hbm, didx, ones, stage, acc, sem):
    cid = lax.axis_index("c")
    sid = lax.axis_index("s")
    base, off, trips = _chunk_span(cid, sid)

    def fill_ones(i, _):
        ones[pl.ds(i * 16, 16)] = jnp.ones((16,), jnp.float32)
        return 0

    lax.fori_loop(0, K // 16, fill_ones, 0)

    def fill_zero(i, _):
        stage[pl.ds(i * 16, 16)] = jnp.zeros((16,), jnp.float32)
        return 0

    lax.fori_loop(0, ROWS_PER_TILE // 16, fill_zero, 0)
    pltpu.sync_copy(stage, acc.at[pl.ds(sid * ROWS_PER_TILE, ROWS_PER_TILE)])
    pltpu.sync_copy(dst_hbm.at[pl.ds(base, CH_MAX)], didx)
    plsc.subcore_barrier()

    def dgroup(g, _):
        descs = [
            pltpu.async_copy(ones, acc.at[didx.at[off + g * DG + i]], sem, add=True)
            for i in range(DG)
        ]
        for d in descs:
            d.wait()
        return 0

    lax.fori_loop(0, trips // DG, dgroup, 0)
    plsc.subcore_barrier()
    pltpu.sync_copy(acc.at[pl.ds(sid * ROWS_PER_TILE, ROWS_PER_TILE)], stage)
    pltpu.sync_copy(stage, out_hbm.at[cid, pl.ds(sid * ROWS_PER_TILE, ROWS_PER_TILE)])


def _sc_scatter16_body(y_hbm, src_hbm, dst_hbm, out_hbm, sidx, didx, rows,
                       stage, acc, *sems):
    cid = lax.axis_index("c")
    sid = lax.axis_index("s")
    base, off, trips = _chunk_span(cid, sid)

    def fill_zero(i, _):
        stage[i, :] = jnp.zeros((16,), jnp.float32)
        return 0

    lax.fori_loop(0, ROWS_PER_TILE, fill_zero, 0)
    pltpu.sync_copy(stage, acc.at[pl.ds(sid * ROWS_PER_TILE, ROWS_PER_TILE)])
    pltpu.sync_copy(src_hbm.at[pl.ds(base, CH_MAX)], sidx)
    pltpu.sync_copy(dst_hbm.at[pl.ds(base, CH_MAX)], didx)
    plsc.subcore_barrier()

    def group(g, _):
        descs = [
            pltpu.async_copy(y_hbm.at[cid].at[sidx.at[off + g * NBUF + b]],
                             rows.at[b], sems[b])
            for b in range(NBUF)
        ]
        for b in range(NBUF):
            descs[b].wait()
            pltpu.sync_copy(rows.at[b], acc.at[didx.at[off + g * NBUF + b]],
                            add=True)
        return 0

    lax.fori_loop(0, trips // NBUF, group, 0)
    plsc.subcore_barrier()
    pltpu.sync_copy(acc.at[pl.ds(sid * ROWS_PER_TILE, ROWS_PER_TILE)], stage)
    pltpu.sync_copy(stage, out_hbm.at[cid, pl.ds(sid * ROWS_PER_TILE, ROWS_PER_TILE)])


@functools.cache
def _sc_kernels():
    mesh = plsc.VectorSubcoreMesh(core_axis_name="c", subcore_axis_name="s",
                                  num_cores=NC, num_subcores=NS)
    params = pltpu.CompilerParams(use_tc_tiling_on_sc=False)
    deg = pl.kernel(
        _sc_degree_body,
        out_type=jax.ShapeDtypeStruct((NC, NPAD), jnp.float32),
        mesh=mesh,
        scratch_types=[
            pltpu.VMEM((CH_MAX, K), jnp.int32),
            pltpu.VMEM((K,), jnp.float32),
            pltpu.VMEM((ROWS_PER_TILE,), jnp.float32),
            pltpu.VMEM_SHARED((NPAD,), jnp.float32),
            pltpu.SemaphoreType.DMA,
        ],
        compiler_params=params,
    )
    scat = pl.kernel(
        _sc_scatter16_body,
        out_type=jax.ShapeDtypeStruct((NC, NPAD, D_HID), jnp.float32),
        mesh=mesh,
        scratch_types=[
            pltpu.VMEM((CH_MAX, K), jnp.int32),
            pltpu.VMEM((CH_MAX, K), jnp.int32),
            pltpu.VMEM((NBUF, K, D_HID), jnp.float32),
            pltpu.VMEM((ROWS_PER_TILE, D_HID), jnp.float32),
            pltpu.VMEM_SHARED((NPAD, D_HID), jnp.float32),
        ] + [pltpu.SemaphoreType.DMA] * NBUF,
        compiler_params=params,
    )
    return deg, scat


_ROWS_BLK = 2000


def _tc1a_body(x_ref, w1_ref, xw_ref):
    xw_ref[...] = jnp.dot(x_ref[...], w1_ref[...],
                          preferred_element_type=jnp.float32)


def _tc1a(x, w1):
    grid = (N // _ROWS_BLK,)
    return pl.pallas_call(
        _tc1a_body,
        grid=grid,
        in_specs=[
            pl.BlockSpec((_ROWS_BLK, D_IN), lambda i: (i, 0)),
            pl.BlockSpec((D_IN, D_HID), lambda i: (0, 0)),
        ],
        out_specs=pl.BlockSpec((_ROWS_BLK, D_HID), lambda i: (i, 0)),
        out_shape=jax.ShapeDtypeStruct((N, D_HID), jnp.float32),
    )(x, w1)


def _tc1b_body(xw_ref, d0_ref, d1_ref, dis_ref, y1_ref):
    deg = d0_ref[...] + d1_ref[...] + 1.0
    dis = lax.rsqrt(deg)
    dis_ref[...] = dis
    y1 = dis * xw_ref[...]
    y1_ref[0] = y1
    y1_ref[1] = y1


def _tc1b(xw, d0, d1):
    grid = (N // _ROWS_BLK,)
    return pl.pallas_call(
        _tc1b_body,
        grid=grid,
        in_specs=[
            pl.BlockSpec((_ROWS_BLK, D_HID), lambda i: (i, 0)),
            pl.BlockSpec((_ROWS_BLK, 1), lambda i: (i, 0)),
            pl.BlockSpec((_ROWS_BLK, 1), lambda i: (i, 0)),
        ],
        out_specs=[
            pl.BlockSpec((_ROWS_BLK, 1), lambda i: (i, 0)),
            pl.BlockSpec((2, _ROWS_BLK, D_HID), lambda i: (0, i, 0)),
        ],
        out_shape=[
            jax.ShapeDtypeStruct((N, 1), jnp.float32),
            jax.ShapeDtypeStruct((2, N, D_HID), jnp.float32),
        ],
    )(xw, d0, d1)


def _pq_specs():
    return pl.BlockSpec((1, _ROWS_BLK, D_HID), lambda i: (0, i, 0)), \
        pl.BlockSpec((1, _ROWS_BLK, D_HID), lambda i: (1, i, 0))


def _tc2_body(p0_ref, p1_ref, y1_ref, dis_ref, b1_ref, z_ref):
    dis = dis_ref[...]
    pre = dis * (p0_ref[0] + p1_ref[0] + y1_ref[0]) + b1_ref[...]
    z = dis * jnp.maximum(pre, 0.0)
    z_ref[0] = z
    z_ref[1] = z


def _tc2(p, y1, dis, b1):
    grid = (N // _ROWS_BLK,)
    blk = pl.BlockSpec((_ROWS_BLK, D_HID), lambda i: (i, 0))
    s0, s1 = _pq_specs()
    yv = pl.BlockSpec((1, _ROWS_BLK, D_HID), lambda i: (0, i, 0))
    return pl.pallas_call(
        _tc2_body,
        grid=grid,
        in_specs=[
            s0, s1, yv,
            pl.BlockSpec((_ROWS_BLK, 1), lambda i: (i, 0)),
            pl.BlockSpec((1, D_HID), lambda i: (0, 0)),
        ],
        out_specs=pl.BlockSpec((2, _ROWS_BLK, D_HID), lambda i: (0, i, 0)),
        out_shape=jax.ShapeDtypeStruct((2, N, D_HID), jnp.float32),
    )(p, p, y1, dis, b1)


def _tc3_body(q0_ref, q1_ref, z_ref, dis_ref, w2_ref, b2_ref, out_ref):
    a = dis_ref[...] * (q0_ref[0] + q1_ref[0] + z_ref[0])
    o = jnp.dot(a, w2_ref[...], preferred_element_type=jnp.float32) + b2_ref[...]
    m = jnp.max(o, axis=1, keepdims=True)
    lse = m + jnp.log(jnp.sum(jnp.exp(o - m), axis=1, keepdims=True))
    out_ref[...] = o - lse


def _tc3(q, z, dis, w2, b2):
    grid = (N // _ROWS_BLK,)
    blk = pl.BlockSpec((_ROWS_BLK, D_HID), lambda i: (i, 0))
    s0, s1 = _pq_specs()
    return pl.pallas_call(
        _tc3_body,
        grid=grid,
        in_specs=[
            s0, s1, pl.BlockSpec((1, _ROWS_BLK, D_HID), lambda i: (0, i, 0)),
            pl.BlockSpec((_ROWS_BLK, 1), lambda i: (i, 0)),
            pl.BlockSpec((D_HID, D_OUT), lambda i: (0, 0)),
            pl.BlockSpec((1, D_OUT), lambda i: (0, 0)),
        ],
        out_specs=pl.BlockSpec((_ROWS_BLK, D_OUT), lambda i: (i, 0)),
        out_shape=jax.ShapeDtypeStruct((N, D_OUT), jnp.float32),
    )(q, q, z, dis, w2, b2)


def kernel(x, edge_index, W1, b1, W2, b2):
    ei = edge_index.astype(jnp.int32)
    pad = EPAD - E
    srcp = jnp.concatenate(
        [ei[0], jnp.zeros((pad,), jnp.int32)]).reshape(TOTAL_CHUNKS, K)
    dstp = jnp.concatenate(
        [ei[1], jnp.full((pad,), NPAD - 1, jnp.int32)]).reshape(TOTAL_CHUNKS, K)

    sc_degree, sc_scatter16 = _sc_kernels()
    deg = sc_degree(dstp)
    xw = _tc1a(x, W1)
    dis, y1 = _tc1b(xw, deg[0, :N, None], deg[1, :N, None])

    p = sc_scatter16(y1, srcp, dstp)
    z = _tc2(p, y1, dis, b1.reshape(1, D_HID))

    q = sc_scatter16(z, srcp, dstp)
    return _tc3(q, z, dis, W2, b2.reshape(1, D_OUT))


# speedup vs baseline: 1.0271x; 1.0271x over previous
import functools

import jax
import jax.numpy as jnp
from jax import lax
from jax.experimental import pallas as pl
from jax.experimental.pallas import tpu as pltpu
from jax.experimental.pallas import tpu_sc as plsc

N = 10000
E = 320000
D_IN = 128
D_HID = 16
D_OUT = 2

NC = 2
NS = 16
NW = NC * NS
NPAD = 10240
ROWS_PER_TILE = NPAD // NS
K = 128
TOTAL_CHUNKS = 2560
EPAD = TOTAL_CHUNKS * K
CH_A = 96
CH_B = 64
CH_MAX = 96
NBUF = 16
DG = 16

def _chunk_span(cid, sid):
    base = lax.select(cid == 0, sid * CH_A, 16 * CH_A + sid * CH_B)
    load_base = jnp.minimum(base, TOTAL_CHUNKS - CH_MAX)
    trips = lax.select(cid == 0, CH_A, CH_B)
    return load_base, base - load_base, trips


def _sc_degree_body(dst_hbm, out_hbm, didx, ones, stage, acc, sem):
    cid = lax.axis_index("c")
    sid = lax.axis_index("s")
    base, off, trips = _chunk_span(cid, sid)

    def fill_ones(i, _):
        ones[pl.ds(i * 16, 16)] = jnp.ones((16,), jnp.float32)
        return 0

    lax.fori_loop(0, K // 16, fill_ones, 0)

    def fill_zero(i, _):
        stage[pl.ds(i * 16, 16)] = jnp.zeros((16,), jnp.float32)
        return 0

    lax.fori_loop(0, ROWS_PER_TILE // 16, fill_zero, 0)
    pltpu.sync_copy(stage, acc.at[pl.ds(sid * ROWS_PER_TILE, ROWS_PER_TILE)])
    pltpu.sync_copy(dst_hbm.at[pl.ds(base, CH_MAX)], didx)
    plsc.subcore_barrier()

    def dgroup(g, _):
        descs = [
            pltpu.async_copy(ones, acc.at[didx.at[off + g * DG + i]], sem, add=True)
            for i in range(DG)
        ]
        for d in descs:
            d.wait()
        return 0

    lax.fori_loop(0, trips // DG, dgroup, 0)
    plsc.subcore_barrier()
    pltpu.sync_copy(acc.at[pl.ds(sid * ROWS_PER_TILE, ROWS_PER_TILE)], stage)
    pltpu.sync_copy(stage, out_hbm.at[cid, pl.ds(sid * ROWS_PER_TILE, ROWS_PER_TILE)])


def _sc_scatter16_body(y_hbm, src_hbm, dst_hbm, out_hbm, sidx, didx, rows,
                       stage, acc, *sems):
    cid = lax.axis_index("c")
    sid = lax.axis_index("s")
    base, off, trips = _chunk_span(cid, sid)

    def fill_zero(i, _):
        stage[i, :] = jnp.zeros((16,), jnp.float32)
        return 0

    lax.fori_loop(0, ROWS_PER_TILE, fill_zero, 0)
    pltpu.sync_copy(stage, acc.at[pl.ds(sid * ROWS_PER_TILE, ROWS_PER_TILE)])
    pltpu.sync_copy(src_hbm.at[pl.ds(base, CH_MAX)], sidx)
    pltpu.sync_copy(dst_hbm.at[pl.ds(base, CH_MAX)], didx)
    plsc.subcore_barrier()

    def group(g, _):
        descs = [
            pltpu.async_copy(y_hbm.at[sidx.at[off + g * NBUF + b]],
                             rows.at[b], sems[b])
            for b in range(NBUF)
        ]
        for b in range(NBUF):
            descs[b].wait()
            pltpu.sync_copy(rows.at[b], acc.at[didx.at[off + g * NBUF + b]],
                            add=True)
        return 0

    lax.fori_loop(0, trips // NBUF, group, 0)
    plsc.subcore_barrier()
    pltpu.sync_copy(acc.at[pl.ds(sid * ROWS_PER_TILE, ROWS_PER_TILE)], stage)
    pltpu.sync_copy(stage, out_hbm.at[cid, pl.ds(sid * ROWS_PER_TILE, ROWS_PER_TILE)])


@functools.cache
def _sc_kernels():
    mesh = plsc.VectorSubcoreMesh(core_axis_name="c", subcore_axis_name="s",
                                  num_cores=NC, num_subcores=NS)
    params = pltpu.CompilerParams(use_tc_tiling_on_sc=False)
    deg = pl.kernel(
        _sc_degree_body,
        out_type=jax.ShapeDtypeStruct((NC, NPAD), jnp.float32),
        mesh=mesh,
        scratch_types=[
            pltpu.VMEM((CH_MAX, K), jnp.int32),
            pltpu.VMEM((K,), jnp.float32),
            pltpu.VMEM((ROWS_PER_TILE,), jnp.float32),
            pltpu.VMEM_SHARED((NPAD,), jnp.float32),
            pltpu.SemaphoreType.DMA,
        ],
        compiler_params=params,
    )
    scat = pl.kernel(
        _sc_scatter16_body,
        out_type=jax.ShapeDtypeStruct((NC, NPAD, D_HID), jnp.float32),
        mesh=mesh,
        scratch_types=[
            pltpu.VMEM((CH_MAX, K), jnp.int32),
            pltpu.VMEM((CH_MAX, K), jnp.int32),
            pltpu.VMEM((NBUF, K, D_HID), jnp.float32),
            pltpu.VMEM((ROWS_PER_TILE, D_HID), jnp.float32),
            pltpu.VMEM_SHARED((NPAD, D_HID), jnp.float32),
        ] + [pltpu.SemaphoreType.DMA] * NBUF,
        compiler_params=params,
    )
    return deg, scat


_ROWS_BLK = 2000


def _tc1a_body(x_ref, w1_ref, xw_ref):
    xw_ref[...] = jnp.dot(x_ref[...], w1_ref[...],
                          preferred_element_type=jnp.float32)


def _tc1a(x, w1):
    grid = (N // _ROWS_BLK,)
    return pl.pallas_call(
        _tc1a_body,
        grid=grid,
        in_specs=[
            pl.BlockSpec((_ROWS_BLK, D_IN), lambda i: (i, 0)),
            pl.BlockSpec((D_IN, D_HID), lambda i: (0, 0)),
        ],
        out_specs=pl.BlockSpec((_ROWS_BLK, D_HID), lambda i: (i, 0)),
        out_shape=jax.ShapeDtypeStruct((N, D_HID), jnp.float32),
    )(x, w1)


def _tc1b_body(xw_ref, d0_ref, d1_ref, dis_ref, y1_ref):
    deg = d0_ref[...] + d1_ref[...] + 1.0
    dis = lax.rsqrt(deg)
    dis_ref[...] = dis
    y1_ref[...] = dis * xw_ref[...]


def _tc1b(xw, d0, d1):
    grid = (N // _ROWS_BLK,)
    return pl.pallas_call(
        _tc1b_body,
        grid=grid,
        in_specs=[
            pl.BlockSpec((_ROWS_BLK, D_HID), lambda i: (i, 0)),
            pl.BlockSpec((_ROWS_BLK, 1), lambda i: (i, 0)),
            pl.BlockSpec((_ROWS_BLK, 1), lambda i: (i, 0)),
        ],
        out_specs=[
            pl.BlockSpec((_ROWS_BLK, 1), lambda i: (i, 0)),
            pl.BlockSpec((_ROWS_BLK, D_HID), lambda i: (i, 0)),
        ],
        out_shape=[
            jax.ShapeDtypeStruct((N, 1), jnp.float32),
            jax.ShapeDtypeStruct((N, D_HID), jnp.float32),
        ],
    )(xw, d0, d1)


def _pq_specs():
    return pl.BlockSpec((1, _ROWS_BLK, D_HID), lambda i: (0, i, 0)), \
        pl.BlockSpec((1, _ROWS_BLK, D_HID), lambda i: (1, i, 0))


def _tc2_body(p0_ref, p1_ref, y1_ref, dis_ref, b1_ref, z_ref):
    dis = dis_ref[...]
    pre = dis * (p0_ref[0] + p1_ref[0] + y1_ref[...]) + b1_ref[...]
    z_ref[...] = dis * jnp.maximum(pre, 0.0)


def _tc2(p, y1, dis, b1):
    grid = (N // _ROWS_BLK,)
    blk = pl.BlockSpec((_ROWS_BLK, D_HID), lambda i: (i, 0))
    s0, s1 = _pq_specs()
    return pl.pallas_call(
        _tc2_body,
        grid=grid,
        in_specs=[
            s0, s1, blk,
            pl.BlockSpec((_ROWS_BLK, 1), lambda i: (i, 0)),
            pl.BlockSpec((1, D_HID), lambda i: (0, 0)),
        ],
        out_specs=blk,
        out_shape=jax.ShapeDtypeStruct((N, D_HID), jnp.float32),
    )(p, p, y1, dis, b1)


def _tc3_body(q0_ref, q1_ref, z_ref, dis_ref, w2_ref, b2_ref, out_ref):
    a = dis_ref[...] * (q0_ref[0] + q1_ref[0] + z_ref[...])
    o = jnp.dot(a, w2_ref[...], preferred_element_type=jnp.float32) + b2_ref[...]
    m = jnp.max(o, axis=1, keepdims=True)
    lse = m + jnp.log(jnp.sum(jnp.exp(o - m), axis=1, keepdims=True))
    out_ref[...] = o - lse


def _tc3(q, z, dis, w2, b2):
    grid = (N // _ROWS_BLK,)
    blk = pl.BlockSpec((_ROWS_BLK, D_HID), lambda i: (i, 0))
    s0, s1 = _pq_specs()
    return pl.pallas_call(
        _tc3_body,
        grid=grid,
        in_specs=[
            s0, s1, blk,
            pl.BlockSpec((_ROWS_BLK, 1), lambda i: (i, 0)),
            pl.BlockSpec((D_HID, D_OUT), lambda i: (0, 0)),
            pl.BlockSpec((1, D_OUT), lambda i: (0, 0)),
        ],
        out_specs=pl.BlockSpec((_ROWS_BLK, D_OUT), lambda i: (i, 0)),
        out_shape=jax.ShapeDtypeStruct((N, D_OUT), jnp.float32),
    )(q, q, z, dis, w2, b2)


def kernel(x, edge_index, W1, b1, W2, b2):
    ei = edge_index.astype(jnp.int32)
    pad = EPAD - E
    srcp = jnp.concatenate(
        [ei[0], jnp.zeros((pad,), jnp.int32)]).reshape(TOTAL_CHUNKS, K)
    dstp = jnp.concatenate(
        [ei[1], jnp.full((pad,), NPAD - 1, jnp.int32)]).reshape(TOTAL_CHUNKS, K)

    sc_degree, sc_scatter16 = _sc_kernels()
    deg = sc_degree(dstp)
    xw = _tc1a(x, W1)
    dis, y1 = _tc1b(xw, deg[0, :N, None], deg[1, :N, None])

    p = sc_scatter16(y1, srcp, dstp)
    z = _tc2(p, y1, dis, b1.reshape(1, D_HID))

    q = sc_scatter16(z, srcp, dstp)
    return _tc3(q, z, dis, W2, b2.reshape(1, D_OUT))
